# baseline (speedup 1.0000x reference)
import functools

import jax
import jax.numpy as jnp
from jax import lax
from jax.experimental import pallas as pl
from jax.experimental.pallas import tpu as pltpu
from jax.experimental.pallas import tpu_sc as plsc

NC = 2
NS = 16
NW = NC * NS
LB = 128
RB = 1024
D = 128
TR = 128


def _mesh():
    return plsc.VectorSubcoreMesh(core_axis_name="c", subcore_axis_name="s")


def _chunks(total):
    out = []
    off = 0
    while off < total:
        cs = min(LB, total - off)
        out.append((off, cs))
        off += cs
    return out


def _make_hist(acc_len, n_batch):
    wpt = acc_len // NS

    @functools.partial(
        pl.kernel,
        out_type=jax.ShapeDtypeStruct((NC * acc_len,), jnp.float32),
        mesh=_mesh(),
        scratch_types=[
            pltpu.VMEM((n_batch, LB), jnp.int32),
            pltpu.VMEM((LB,), jnp.float32),
            pltpu.VMEM_SHARED((acc_len,), jnp.float32),
        ],
    )
    def hist(idx_hbm, zeros_hbm, out_hbm, idx_v, ones_v, acc):
        cid = lax.axis_index("c")
        sid = lax.axis_index("s")
        wid = sid * NC + cid
        base = sid * wpt
        pltpu.sync_copy(zeros_hbm.at[pl.ds(base, wpt)], acc.at[pl.ds(base, wpt)])
        pltpu.sync_copy(idx_hbm.at[wid], idx_v)
        for i in range(LB // 16):
            ones_v[pl.ds(i * 16, 16)] = jnp.ones((16,), jnp.float32)
        plsc.subcore_barrier()

        def body(j, carry):
            pltpu.sync_copy(ones_v, acc.at[idx_v.at[j]], add=True)
            return carry

        lax.fori_loop(0, n_batch, body, 0)
        plsc.subcore_barrier()
        pltpu.sync_copy(acc.at[pl.ds(base, wpt)],
                        out_hbm.at[pl.ds(cid * acc_len + base, wpt)])

    return hist


def _make_seg_sum(n_acc, n_batch):
    half = n_acc // NC
    zpt = (half + TR) // NS
    wpt = half // NS
    NB = 2

    @functools.partial(
        pl.kernel,
        out_type=jax.ShapeDtypeStruct((n_acc, D), jnp.float32),
        mesh=_mesh(),
        scratch_types=(
            [pltpu.VMEM((n_batch, LB), jnp.int32)] * 2
            + [pltpu.VMEM((LB, D), jnp.float32)] * NB
            + [pltpu.VMEM_SHARED((half + TR, D), jnp.float32)]
            + [pltpu.SemaphoreType.DMA] * (2 * NB)
        ),
    )
    def seg(table_hbm, src_hbm, dst_hbm, zeros_hbm, out_hbm,
            idx_s, idx_d, *bufs_sems):
        rbufs = bufs_sems[:NB]
        acc = bufs_sems[NB]
        gsems = bufs_sems[NB + 1:2 * NB + 1]
        ssems = bufs_sems[2 * NB + 1:]
        cid = lax.axis_index("c")
        sid = lax.axis_index("s")
        pltpu.sync_copy(zeros_hbm, rbufs[0])
        for off, cs in _chunks(zpt):
            pltpu.sync_copy(rbufs[0].at[pl.ds(0, cs)],
                            acc.at[pl.ds(sid * zpt + off, cs)])
        pltpu.sync_copy(src_hbm.at[sid], idx_s)
        pltpu.sync_copy(dst_hbm.at[cid, sid], idx_d)
        plsc.subcore_barrier()

        def wait_gather(b):
            pltpu.make_async_copy(
                table_hbm.at[idx_s.at[0]], rbufs[b], gsems[b]).wait()

        def wait_scatter(b):
            pltpu.make_async_copy(
                rbufs[b], acc.at[idx_d.at[0]], ssems[b]).wait()

        for b in range(NB):
            pltpu.async_copy(table_hbm.at[idx_s.at[b]], rbufs[b], gsems[b])

        def body(g, carry):
            for b in range(NB):
                j = NB * g + b
                wait_gather(b)
                pltpu.async_copy(rbufs[b], acc.at[idx_d.at[j]], ssems[b],
                                 add=True)
            for b in range(NB):
                j2 = NB * (g + 1) + b
                wait_scatter(b)
                pltpu.async_copy(table_hbm.at[idx_s.at[j2]], rbufs[b],
                                 gsems[b])
            return carry

        lax.fori_loop(0, n_batch // NB - 1, body, 0)
        g_last = n_batch // NB - 1
        for b in range(NB):
            wait_gather(b)
            pltpu.async_copy(rbufs[b], acc.at[idx_d.at[NB * g_last + b]],
                             ssems[b], add=True)
        for b in range(NB):
            wait_scatter(b)
        plsc.subcore_barrier()
        for off, cs in _chunks(wpt):
            pltpu.sync_copy(acc.at[pl.ds(TR + sid * wpt + off, cs)],
                            rbufs[0].at[pl.ds(0, cs)])
            pltpu.sync_copy(rbufs[0].at[pl.ds(0, cs)],
                            out_hbm.at[pl.ds(cid * half + sid * wpt + off, cs)])

    return seg


def _norm_from(deg_ref):
    deg = jnp.sum(deg_ref[...], axis=1, keepdims=True)
    return lax.rsqrt(jnp.maximum(deg, 1.0))


def _tc1_body(deg_ref, lg_ref, ft_ref, wp_ref, bp_ref, w1a_ref, w1b_ref,
              o_ref):
    no = _norm_from(deg_ref)
    feat = jnp.dot(ft_ref[...], wp_ref[...],
                   preferred_element_type=jnp.float32) + bp_ref[...]
    a = jnp.dot(lg_ref[...] * no, w1a_ref[...],
                preferred_element_type=jnp.float32)
    b = jnp.dot(feat * no, w1b_ref[...], preferred_element_type=jnp.float32)
    o_ref[...] = a + b


def _tc2_body(agg_ref, degi_ref, dego_ref, b1_ref, w2_ref, o_ref):
    ni = _norm_from(degi_ref)
    no = _norm_from(dego_ref)
    h = w2_ref.shape[0]
    x = agg_ref[:, :h] * ni + b1_ref[...]
    x = jnp.maximum(x, 0.0) * no
    o_ref[...] = jnp.dot(x, w2_ref[...], preferred_element_type=jnp.float32)


def _tc3_body(agg_ref, degi_ref, b2_ref, o_ref):
    ni = _norm_from(degi_ref)
    h = b2_ref.shape[1]
    o_ref[...] = agg_ref[:, :h] * ni + b2_ref[...]


def kernel(logits, features, edge_index, W_proj, b_proj, W1, b1, W2, b2):
    n, n_cls = logits.shape
    fdim = features.shape[1]
    fh = W_proj.shape[1]
    hid = W1.shape[1]
    out_dim = W2.shape[1]
    e = edge_index.shape[1]

    grid = -(-n // RB)
    n_acc = grid * RB
    half = n_acc // NC

    src = edge_index[0]
    dst = edge_index[1]

    capc = NS * LB
    nbc = -(-e // capc)
    nbc += (-nbc) % 4
    padc = nbc * capc - e
    src2 = jnp.concatenate(
        [src, jnp.zeros((padc,), jnp.int32)]).reshape(NS, nbc, LB)
    dloc = []
    for c in range(NC):
        lo = c * half
        in_rng = (dst >= lo) & (dst < lo + half)
        loc = jnp.where(in_rng, dst - lo + TR, dst % TR)
        dloc.append(jnp.concatenate([loc, jnp.zeros((padc,), jnp.int32)]))
    dst2 = jnp.stack(dloc).reshape(NC, NS, nbc, LB)

    cap = NW * LB
    acc_len = 2 * n_acc
    nb2 = -(-2 * e // cap)
    nb2 += nb2 % 2
    pad2 = nb2 * cap - 2 * e
    both = jnp.concatenate([src, dst + n_acc, jnp.full((pad2,), n, jnp.int32)])
    idx2 = both.reshape(NW, nb2, LB)

    zeros_flat = jnp.zeros((acc_len,), jnp.float32)
    hist_parts = _make_hist(acc_len, nb2)(idx2, zeros_flat)
    hp = hist_parts.reshape(NC, acc_len)
    deg_out_p = hp[:, :n_acc].T
    deg_in_p = hp[:, n_acc:].T

    w1a = jnp.pad(W1[:n_cls], ((0, 0), (0, D - hid)))
    w1b = jnp.pad(W1[n_cls:], ((0, 0), (0, D - hid)))
    z1 = pl.pallas_call(
        _tc1_body,
        grid=(grid,),
        in_specs=[
            pl.BlockSpec((RB, NC), lambda i: (i, 0)),
            pl.BlockSpec((RB, n_cls), lambda i: (i, 0)),
            pl.BlockSpec((RB, fdim), lambda i: (i, 0)),
            pl.BlockSpec((fdim, fh), lambda i: (0, 0)),
            pl.BlockSpec((1, fh), lambda i: (0, 0)),
            pl.BlockSpec((n_cls, D), lambda i: (0, 0)),
            pl.BlockSpec((fh, D), lambda i: (0, 0)),
        ],
        out_specs=pl.BlockSpec((RB, D), lambda i: (i, 0)),
        out_shape=jax.ShapeDtypeStruct((n_acc, D), jnp.float32),
    )(deg_out_p, logits, features, W_proj, b_proj.reshape(1, fh), w1a, w1b)

    zeros_nd = jnp.zeros((LB, D), jnp.float32)
    seg = _make_seg_sum(n_acc, nbc)
    agg1 = seg(z1, src2, dst2, zeros_nd)

    w2p = jnp.pad(W2, ((0, 0), (0, D - out_dim)))
    b1r = b1.reshape(1, hid)
    z2 = pl.pallas_call(
        _tc2_body,
        grid=(grid,),
        in_specs=[
            pl.BlockSpec((RB, D), lambda i: (i, 0)),
            pl.BlockSpec((RB, NC), lambda i: (i, 0)),
            pl.BlockSpec((RB, NC), lambda i: (i, 0)),
            pl.BlockSpec((1, hid), lambda i: (0, 0)),
            pl.BlockSpec((hid, D), lambda i: (0, 0)),
        ],
        out_specs=pl.BlockSpec((RB, D), lambda i: (i, 0)),
        out_shape=jax.ShapeDtypeStruct((n_acc, D), jnp.float32),
    )(agg1, deg_in_p, deg_out_p, b1r, w2p)

    agg2 = seg(z2, src2, dst2, zeros_nd)

    b2r = jnp.pad(b2, (0, hid - out_dim)).reshape(1, hid)
    outp = pl.pallas_call(
        _tc3_body,
        grid=(grid,),
        in_specs=[
            pl.BlockSpec((RB, D), lambda i: (i, 0)),
            pl.BlockSpec((RB, NC), lambda i: (i, 0)),
            pl.BlockSpec((1, hid), lambda i: (0, 0)),
        ],
        out_specs=pl.BlockSpec((RB, hid), lambda i: (i, 0)),
        out_shape=jax.ShapeDtypeStruct((n_acc, hid), jnp.float32),
    )(agg2, deg_in_p, b2r)

    return outp[:n, :out_dim]
